# baseline (speedup 1.0000x reference)
import functools

import jax
import jax.numpy as jnp
from jax import lax
from jax.experimental import pallas as pl
from jax.experimental.pallas import tpu as pltpu
from jax.experimental.pallas import tpu_sc as plsc

_B = 16384
_D = 64
_L = 16


@functools.lru_cache(maxsize=None)
def _build_gather():
    info = plsc.get_sparse_core_info()
    nc, ns = info.num_cores, info.num_subcores
    nw = nc * ns
    b_per_w = _B // nw
    n_chunks = b_per_w // _L
    mesh = plsc.VectorSubcoreMesh(core_axis_name="c", subcore_axis_name="s")

    @functools.partial(
        pl.kernel,
        mesh=mesh,
        out_type=jax.ShapeDtypeStruct((_B // 2, 2 * _D), jnp.float32),
        scratch_types=[
            pltpu.VMEM((b_per_w,), jnp.int32),
            pltpu.VMEM((b_per_w,), jnp.int32),
            pltpu.VMEM((b_per_w, 2 * _D), jnp.float32),
            pltpu.VMEM((b_per_w // 2, 2 * _D), jnp.float32),
            pltpu.SemaphoreType.DMA,
        ],
    )
    def gather(ids_hbm, table_hbm, out_hbm, idx_v, pidx_v, rows_v, cmp_v, sem):
        wid = lax.axis_index("s") * nc + lax.axis_index("c")
        base = wid * b_per_w
        pltpu.sync_copy(ids_hbm.at[pl.ds(base, b_per_w)], idx_v)

        def mk_pairs(c, carry):
            v = idx_v[pl.ds(c * _L, _L)]
            pidx_v[pl.ds(c * _L, _L)] = lax.shift_right_logical(v, 1)
            return carry

        lax.fori_loop(0, n_chunks, mk_pairs, 0)
        pltpu.async_copy(table_hbm.at[pidx_v], rows_v, sem).wait()

        def select(c, carry):
            v = idx_v[pl.ds(c * _L, _L)]
            par = (v & 1) * _D
            for j in range(_L):
                row = c * _L + j
                row2 = c * (_L // 2) + j // 2
                off = par[j]
                dst_col = (j & 1) * _D
                for k in range(0, _D, _L):
                    cmp_v[row2, pl.ds(dst_col + k, _L)] = (
                        rows_v[row, pl.ds(off + k, _L)]
                    )
            return carry

        lax.fori_loop(0, n_chunks, select, 0)
        pltpu.sync_copy(cmp_v, out_hbm.at[pl.ds(wid * (b_per_w // 2), b_per_w // 2)])

    return gather


def kernel(ids, table):
    out = _build_gather()(ids.astype(jnp.int32), table.reshape(500000, 2 * _D))
    return out.reshape(_B, _D)


# speedup vs baseline: 1.7116x; 1.7116x over previous
import functools

import jax
import jax.numpy as jnp
from jax import lax
from jax.experimental import pallas as pl
from jax.experimental.pallas import tpu as pltpu
from jax.experimental.pallas import tpu_sc as plsc

_B = 16384
_D = 64
_L = 16
_K = 16


@functools.lru_cache(maxsize=None)
def _build_gather():
    info = plsc.get_sparse_core_info()
    nc, ns = info.num_cores, info.num_subcores
    nw = nc * ns
    b_per_w = _B // nw
    n_chunks = b_per_w // _K
    mesh = plsc.VectorSubcoreMesh(core_axis_name="c", subcore_axis_name="s")

    @functools.partial(
        pl.kernel,
        mesh=mesh,
        out_type=jax.ShapeDtypeStruct((_B, _D), jnp.float32),
        scratch_types=[
            pltpu.VMEM((b_per_w,), jnp.int32),
            pltpu.VMEM((b_per_w, _D), jnp.float32),
            pltpu.SemaphoreType.DMA,
        ],
    )
    def gather(ids_hbm, table_hbm, out_hbm, idx_v, rows_v, sem):
        wid = lax.axis_index("s") * nc + lax.axis_index("c")
        base = wid * b_per_w
        pltpu.sync_copy(ids_hbm.at[pl.ds(base, b_per_w)], idx_v)

        def fire(c):
            cb = c * _K
            vec = idx_v[pl.ds(cb, _K)]
            for j in range(_K):
                pltpu.async_copy(
                    table_hbm.at[pl.ds(vec[j], 1), :],
                    rows_v.at[pl.ds(cb + j, 1), :],
                    sem,
                )

        def drain(c):
            pltpu.make_async_copy(
                table_hbm.at[pl.ds(0, _K), :],
                rows_v.at[pl.ds(c * _K, _K), :],
                sem,
            ).wait()

        def step(c, carry):
            fire(c)
            drain(c - 1)
            return carry

        fire(0)
        lax.fori_loop(1, n_chunks, step, 0)
        drain(n_chunks - 1)

        pltpu.sync_copy(rows_v, out_hbm.at[pl.ds(base, b_per_w)])

    return gather


def kernel(ids, table):
    return _build_gather()(ids.astype(jnp.int32), table)
